# Initial kernel scaffold; baseline (speedup 1.0000x reference)
import functools

import jax
import jax.numpy as jnp
from jax import lax
from jax.experimental import pallas as pl
from jax.experimental.pallas import tpu as pltpu
from jax.experimental.pallas import tpu_sc as plsc

N0, N1, N2 = 10000, 5000, 2500
D_IN, D_HID, D_OUT = 256, 256, 128
E0, E1 = 160000, 80000

NC, NS = 2, 16
NW = NC * NS
CHUNK = 128

N1P = 5120
N2P = 2560


def _make_agg(n_tgt_pad, d, n_chunks):
    mesh = plsc.VectorSubcoreMesh(core_axis_name="c", subcore_axis_name="s")
    rz = n_tgt_pad // NS

    @functools.partial(
        pl.kernel,
        mesh=mesh,
        out_type=[
            jax.ShapeDtypeStruct((NC, n_tgt_pad, d), jnp.float32),
            jax.ShapeDtypeStruct((NC, n_tgt_pad, 16), jnp.float32),
        ],
        scratch_types=[
            pltpu.VMEM((n_chunks, CHUNK), jnp.int32),
            pltpu.VMEM((n_chunks, CHUNK), jnp.int32),
            pltpu.VMEM((CHUNK, d), jnp.float32),
            pltpu.VMEM((CHUNK, 16), jnp.float32),
            pltpu.VMEM_SHARED((n_tgt_pad, d), jnp.float32),
            pltpu.VMEM_SHARED((n_tgt_pad, 16), jnp.float32),
            pltpu.SemaphoreType.DMA,
        ],
    )
    def agg(table, src, dst, zeros_d, zeros_c, ones, sums, cnts,
            src_v, dst_v, rows_v, ones_v, sum_sh, cnt_sh, sem):
        c = lax.axis_index("c")
        s = lax.axis_index("s")
        wid = c * NS + s
        pltpu.sync_copy(src.at[wid], src_v)
        pltpu.sync_copy(dst.at[wid], dst_v)
        pltpu.sync_copy(ones, ones_v)
        pltpu.sync_copy(zeros_d.at[pl.ds(s * rz, rz)], sum_sh.at[pl.ds(s * rz, rz)])
        pltpu.sync_copy(zeros_c.at[pl.ds(s * rz, rz)], cnt_sh.at[pl.ds(s * rz, rz)])
        plsc.subcore_barrier()

        def chunk(j, carry):
            pltpu.async_copy(table.at[src_v.at[j]], rows_v, sem).wait()
            pltpu.sync_copy(rows_v, sum_sh.at[dst_v.at[j]], add=True)
            pltpu.sync_copy(ones_v, cnt_sh.at[dst_v.at[j]], add=True)
            return carry

        lax.fori_loop(0, n_chunks, chunk, 0)
        plsc.subcore_barrier()
        pltpu.sync_copy(sum_sh.at[pl.ds(s * rz, rz)], sums.at[c, pl.ds(s * rz, rz)])
        pltpu.sync_copy(cnt_sh.at[pl.ds(s * rz, rz)], cnts.at[c, pl.ds(s * rz, rz)])

    return agg


def _prep_edges(ei, e_pad, n_src_mod, n_tgt, n_tgt_pad):
    e = ei.shape[1]
    ar = jnp.arange(e_pad - e, dtype=jnp.int32)
    src = jnp.concatenate([ei[0].astype(jnp.int32), ar % n_src_mod])
    dst = jnp.concatenate([ei[1].astype(jnp.int32), n_tgt + ar % (n_tgt_pad - n_tgt)])
    return src.reshape(NW, -1, CHUNK), dst.reshape(NW, -1, CHUNK)


def _layer0_tc(p0, p1, c0, c1, x_tgt, wl, bl, wr, wp):
    blk = 512
    grid = (N1P // blk,)

    def body(p0_r, p1_r, c0_r, c1_r, xt_r, wl_r, bl_r, wr_r, wp_r, h_r, pp_r):
        cnt = jnp.maximum(c0_r[:, :1] + c1_r[:, :1], 1.0)
        mean = (p0_r[:] + p1_r[:]) / cnt
        h = jnp.dot(mean, wl_r[:], preferred_element_type=jnp.float32)
        h = h + bl_r[:] + jnp.dot(xt_r[:], wr_r[:], preferred_element_type=jnp.float32)
        h = jnp.maximum(h, 0.0)
        h_r[:] = h
        pp_r[:] = jnp.dot(h, wp_r[:], preferred_element_type=jnp.float32)

    row_spec = lambda d: pl.BlockSpec((blk, d), lambda i: (i, 0))
    full_spec = lambda a, b: pl.BlockSpec((a, b), lambda i: (0, 0))
    return pl.pallas_call(
        body,
        grid=grid,
        in_specs=[
            row_spec(D_HID), row_spec(D_HID), row_spec(16), row_spec(16),
            row_spec(D_IN),
            full_spec(D_IN, D_HID), full_spec(1, D_HID), full_spec(D_IN, D_HID),
            full_spec(D_HID, D_OUT),
        ],
        out_specs=[row_spec(D_HID), row_spec(D_OUT)],
        out_shape=[
            jax.ShapeDtypeStruct((N1P, D_HID), jnp.float32),
            jax.ShapeDtypeStruct((N1P, D_OUT), jnp.float32),
        ],
    )(p0, p1, c0, c1, x_tgt, wl, bl, wr, wp)


def _layer1_tc(p0, p1, c0, c1, h_tgt, wr, bl):
    blk = 512
    grid = (N2P // blk,)

    def body(p0_r, p1_r, c0_r, c1_r, ht_r, wr_r, bl_r, o_r):
        cnt = jnp.maximum(c0_r[:, :1] + c1_r[:, :1], 1.0)
        mean = (p0_r[:] + p1_r[:]) / cnt
        z = mean + bl_r[:] + jnp.dot(ht_r[:], wr_r[:], preferred_element_type=jnp.float32)
        z = z - jnp.max(z, axis=-1, keepdims=True)
        o_r[:] = z - jnp.log(jnp.sum(jnp.exp(z), axis=-1, keepdims=True))

    row_spec = lambda d: pl.BlockSpec((blk, d), lambda i: (i, 0))
    full_spec = lambda a, b: pl.BlockSpec((a, b), lambda i: (0, 0))
    return pl.pallas_call(
        body,
        grid=grid,
        in_specs=[
            row_spec(D_OUT), row_spec(D_OUT), row_spec(16), row_spec(16),
            row_spec(D_HID),
            full_spec(D_HID, D_OUT), full_spec(1, D_OUT),
        ],
        out_specs=row_spec(D_OUT),
        out_shape=jax.ShapeDtypeStruct((N2P, D_OUT), jnp.float32),
    )(p0, p1, c0, c1, h_tgt, wr, bl)


def kernel(x, edge_index0, edge_index1, W_l0, b_l0, W_r0, W_l1, b_l1, W_r1):
    e0_pad = NW * 40 * CHUNK
    e1_pad = NW * 20 * CHUNK
    src0, dst0 = _prep_edges(edge_index0, e0_pad, N1, N1, N1P)
    src1, dst1 = _prep_edges(edge_index1, e1_pad, N2, N2, N2P)

    ones = jnp.ones((CHUNK, 16), jnp.float32)
    z0d = jnp.zeros((N1P, D_HID), jnp.float32)
    z0c = jnp.zeros((N1P, 16), jnp.float32)
    z1d = jnp.zeros((N2P, D_OUT), jnp.float32)
    z1c = jnp.zeros((N2P, 16), jnp.float32)

    agg0 = _make_agg(N1P, D_HID, 40)
    sums0, cnts0 = agg0(x, src0, dst0, z0d, z0c, ones)

    x_tgt = jnp.zeros((N1P, D_IN), jnp.float32).at[:N1].set(x[:N1])
    h, p_proj = _layer0_tc(
        sums0[0], sums0[1], cnts0[0], cnts0[1], x_tgt,
        W_l0.T, b_l0.reshape(1, -1), W_r0.T, W_l1.T)

    agg1 = _make_agg(N2P, D_OUT, 20)
    sums1, cnts1 = agg1(p_proj, src1, dst1, z1d, z1c, ones)

    out = _layer1_tc(
        sums1[0], sums1[1], cnts1[0], cnts1[1], h[:N2P],
        W_r1.T, b_l1.reshape(1, -1))
    return out[:N2]


# speedup vs baseline: 4.9911x; 4.9911x over previous
import functools

import jax
import jax.numpy as jnp
from jax import lax
from jax.experimental import pallas as pl
from jax.experimental.pallas import tpu as pltpu
from jax.experimental.pallas import tpu_sc as plsc

N0, N1, N2 = 10000, 5000, 2500
D_IN, D_HID, D_OUT = 256, 256, 128
E0, E1 = 160000, 80000

NC, NS = 2, 16
NW = NC * NS

N1P = 5120
N2P = 2560
D0A = 272
D1A = 144
CH0, NCH0 = 32, 160
CH1, NCH1 = 64, 40


def _make_agg(n_tgt_pad, d, n_chunks, chunk):
    mesh = plsc.VectorSubcoreMesh(core_axis_name="c", subcore_axis_name="s")
    rz = n_tgt_pad // NS

    @functools.partial(
        pl.kernel,
        mesh=mesh,
        out_type=jax.ShapeDtypeStruct((NC, n_tgt_pad, d), jnp.float32),
        compiler_params=pltpu.CompilerParams(use_tc_tiling_on_sc=False),
        scratch_types=[
            pltpu.VMEM((n_chunks, chunk), jnp.int32),
            pltpu.VMEM((n_chunks, chunk), jnp.int32),
            pltpu.VMEM((chunk, d), jnp.float32),
            pltpu.VMEM_SHARED((n_tgt_pad, d), jnp.float32),
            pltpu.SemaphoreType.DMA,
        ],
    )
    def agg(table, src, dst, zeros, sums, src_v, dst_v, rows_v, sum_sh, sem):
        c = lax.axis_index("c")
        s = lax.axis_index("s")
        wid = c * NS + s
        pltpu.sync_copy(src.at[wid], src_v)
        pltpu.sync_copy(dst.at[wid], dst_v)
        pltpu.sync_copy(zeros.at[pl.ds(s * rz, rz)], sum_sh.at[pl.ds(s * rz, rz)])
        plsc.subcore_barrier()

        def chunk_body(j, carry):
            pltpu.async_copy(table.at[src_v.at[j]], rows_v, sem).wait()
            pltpu.sync_copy(rows_v, sum_sh.at[dst_v.at[j]], add=True)
            return carry

        lax.fori_loop(0, n_chunks, chunk_body, 0)
        plsc.subcore_barrier()
        pltpu.sync_copy(sum_sh.at[pl.ds(s * rz, rz)], sums.at[c, pl.ds(s * rz, rz)])

    return agg


def _prep_edges(ei, n_chunks, chunk, n_src_mod, n_tgt, n_tgt_pad):
    e = ei.shape[1]
    e_pad = NW * n_chunks * chunk
    ar = jnp.arange(e_pad - e, dtype=jnp.int32)
    src = jnp.concatenate([ei[0].astype(jnp.int32), ar % n_src_mod])
    dst = jnp.concatenate([ei[1].astype(jnp.int32), n_tgt + ar % (n_tgt_pad - n_tgt)])
    return src.reshape(NW, n_chunks, chunk), dst.reshape(NW, n_chunks, chunk)


def _proj0_tc(xpad, wl, wr):
    blk = 640
    grid = (N1P // blk,)

    def body(x_r, wl_r, wr_r, xa_r, r0_r):
        xa_r[:, :D_HID] = jnp.dot(x_r[:], wl_r[:],
                                  preferred_element_type=jnp.float32)
        xa_r[:, D_HID:D_HID + 1] = jnp.ones((blk, 1), jnp.float32)
        xa_r[:, D_HID + 1:] = jnp.zeros((blk, D0A - D_HID - 1), jnp.float32)
        r0_r[:] = jnp.dot(x_r[:], wr_r[:], preferred_element_type=jnp.float32)

    return pl.pallas_call(
        body,
        grid=grid,
        in_specs=[
            pl.BlockSpec((blk, D_IN), lambda i: (i, 0)),
            pl.BlockSpec((D_IN, D_HID), lambda i: (0, 0)),
            pl.BlockSpec((D_IN, D_HID), lambda i: (0, 0)),
        ],
        out_specs=[
            pl.BlockSpec((blk, D0A), lambda i: (i, 0)),
            pl.BlockSpec((blk, D_HID), lambda i: (i, 0)),
        ],
        out_shape=[
            jax.ShapeDtypeStruct((N1P, D0A), jnp.float32),
            jax.ShapeDtypeStruct((N1P, D_HID), jnp.float32),
        ],
    )(xpad, wl, wr)


def _mid_tc(p0, p1, r0, bl0, wp, wr1):
    blk = 640
    grid = (N1P // blk,)

    def body(p0_r, p1_r, r0_r, bl_r, wp_r, wr_r, xa_r, r1_r):
        cnt = jnp.maximum(p0_r[:, D_HID:D_HID + 1] + p1_r[:, D_HID:D_HID + 1],
                          1.0)
        mean = (p0_r[:, :D_HID] + p1_r[:, :D_HID]) / cnt
        h = jnp.maximum(mean + bl_r[:] + r0_r[:], 0.0)
        xa_r[:, :D_OUT] = jnp.dot(h, wp_r[:], preferred_element_type=jnp.float32)
        xa_r[:, D_OUT:D_OUT + 1] = jnp.ones((blk, 1), jnp.float32)
        xa_r[:, D_OUT + 1:] = jnp.zeros((blk, D1A - D_OUT - 1), jnp.float32)
        r1_r[:] = jnp.dot(h, wr_r[:], preferred_element_type=jnp.float32)

    return pl.pallas_call(
        body,
        grid=grid,
        in_specs=[
            pl.BlockSpec((blk, D0A), lambda i: (i, 0)),
            pl.BlockSpec((blk, D0A), lambda i: (i, 0)),
            pl.BlockSpec((blk, D_HID), lambda i: (i, 0)),
            pl.BlockSpec((1, D_HID), lambda i: (0, 0)),
            pl.BlockSpec((D_HID, D_OUT), lambda i: (0, 0)),
            pl.BlockSpec((D_HID, D_OUT), lambda i: (0, 0)),
        ],
        out_specs=[
            pl.BlockSpec((blk, D1A), lambda i: (i, 0)),
            pl.BlockSpec((blk, D_OUT), lambda i: (i, 0)),
        ],
        out_shape=[
            jax.ShapeDtypeStruct((N1P, D1A), jnp.float32),
            jax.ShapeDtypeStruct((N1P, D_OUT), jnp.float32),
        ],
    )(p0, p1, r0, bl0, wp, wr1)


def _final_tc(p0, p1, r1, bl1):
    blk = 640
    grid = (N2P // blk,)

    def body(p0_r, p1_r, r1_r, bl_r, o_r):
        cnt = jnp.maximum(p0_r[:, D_OUT:D_OUT + 1] + p1_r[:, D_OUT:D_OUT + 1],
                          1.0)
        mean = (p0_r[:, :D_OUT] + p1_r[:, :D_OUT]) / cnt
        z = mean + bl_r[:] + r1_r[:]
        z = z - jnp.max(z, axis=-1, keepdims=True)
        o_r[:] = z - jnp.log(jnp.sum(jnp.exp(z), axis=-1, keepdims=True))

    return pl.pallas_call(
        body,
        grid=grid,
        in_specs=[
            pl.BlockSpec((blk, D1A), lambda i: (i, 0)),
            pl.BlockSpec((blk, D1A), lambda i: (i, 0)),
            pl.BlockSpec((blk, D_OUT), lambda i: (i, 0)),
            pl.BlockSpec((1, D_OUT), lambda i: (0, 0)),
        ],
        out_specs=pl.BlockSpec((blk, D_OUT), lambda i: (i, 0)),
        out_shape=jax.ShapeDtypeStruct((N2P, D_OUT), jnp.float32),
    )(p0, p1, r1, bl1)


def kernel(x, edge_index0, edge_index1, W_l0, b_l0, W_r0, W_l1, b_l1, W_r1):
    src0, dst0 = _prep_edges(edge_index0, NCH0, CH0, N1, N1, N1P)
    src1, dst1 = _prep_edges(edge_index1, NCH1, CH1, N2, N2, N2P)

    z0 = jnp.zeros((N1P, D0A), jnp.float32)
    z1 = jnp.zeros((N2P, D1A), jnp.float32)

    xpad = jnp.zeros((N1P, D_IN), jnp.float32).at[:N1].set(x[:N1])
    xa0, r0 = _proj0_tc(xpad, W_l0.T, W_r0.T)

    agg0 = _make_agg(N1P, D0A, NCH0, CH0)
    sums0 = agg0(xa0, src0, dst0, z0)

    xa1, r1 = _mid_tc(sums0[0], sums0[1], r0, b_l0.reshape(1, -1),
                      W_l1.T, W_r1.T)

    agg1 = _make_agg(N2P, D1A, NCH1, CH1)
    sums1 = agg1(xa1, src1, dst1, z1)

    out = _final_tc(sums1[0], sums1[1], r1[:N2P], b_l1.reshape(1, -1))
    return out[:N2]


# speedup vs baseline: 6.8969x; 1.3818x over previous
import functools

import jax
import jax.numpy as jnp
from jax import lax
from jax.experimental import pallas as pl
from jax.experimental.pallas import tpu as pltpu
from jax.experimental.pallas import tpu_sc as plsc

N0, N1, N2 = 10000, 5000, 2500
D_IN, D_HID, D_OUT = 256, 256, 128
E0, E1 = 160000, 80000

NC, NS = 2, 16
NW = NC * NS

N1P = 5120
N2P = 2560
D0A = 272
D1A = 144
CH0, NCH0 = 32, 160
CH1, NCH1 = 64, 40


def _make_agg(n_tgt_pad, d, n_chunks, chunk):
    mesh = plsc.VectorSubcoreMesh(core_axis_name="c", subcore_axis_name="s")
    rz = n_tgt_pad // NS

    @functools.partial(
        pl.kernel,
        mesh=mesh,
        out_type=jax.ShapeDtypeStruct((NC, n_tgt_pad, d), jnp.float32),
        compiler_params=pltpu.CompilerParams(use_tc_tiling_on_sc=False),
        scratch_types=[
            pltpu.VMEM((n_chunks, chunk), jnp.int32),
            pltpu.VMEM((n_chunks, chunk), jnp.int32),
            pltpu.VMEM((chunk, d), jnp.float32),
            pltpu.VMEM((chunk, d), jnp.float32),
            pltpu.VMEM_SHARED((n_tgt_pad, d), jnp.float32),
            pltpu.SemaphoreType.DMA,
        ],
    )
    def agg(table, src, dst, zeros, sums, src_v, dst_v, rows0, rows1, sum_sh,
            sem):
        c = lax.axis_index("c")
        s = lax.axis_index("s")
        wid = c * NS + s
        pltpu.sync_copy(src.at[wid], src_v)
        pltpu.sync_copy(dst.at[wid], dst_v)
        pltpu.sync_copy(zeros.at[pl.ds(s * rz, rz)], sum_sh.at[pl.ds(s * rz, rz)])
        plsc.subcore_barrier()

        pltpu.async_copy(table.at[src_v.at[0]], rows0, sem)

        def pair_body(jj, carry):
            for parity, cur, nxt in ((0, rows0, rows1), (1, rows1, rows0)):
                j = 2 * jj + parity
                jn = (j + 1) % n_chunks
                pltpu.async_copy(table.at[src_v.at[jn]], nxt, sem)
                pltpu.make_async_copy(table.at[src_v.at[j]], cur, sem).wait()
                pltpu.sync_copy(cur, sum_sh.at[dst_v.at[j]], add=True)
            return carry

        lax.fori_loop(0, n_chunks // 2, pair_body, 0)
        pltpu.make_async_copy(table.at[src_v.at[0]], rows0, sem).wait()
        plsc.subcore_barrier()
        pltpu.sync_copy(sum_sh.at[pl.ds(s * rz, rz)], sums.at[c, pl.ds(s * rz, rz)])

    return agg


def _prep_edges(ei, n_chunks, chunk, n_src_mod, n_tgt, n_tgt_pad):
    e = ei.shape[1]
    e_pad = NW * n_chunks * chunk
    ar = jnp.arange(e_pad - e, dtype=jnp.int32)
    src = jnp.concatenate([ei[0].astype(jnp.int32), ar % n_src_mod])
    dst = jnp.concatenate([ei[1].astype(jnp.int32), n_tgt + ar % (n_tgt_pad - n_tgt)])
    return src.reshape(NW, n_chunks, chunk), dst.reshape(NW, n_chunks, chunk)


def _proj0_tc(xpad, wl, wr):
    blk = 640
    grid = (N1P // blk,)

    def body(x_r, wl_r, wr_r, xa_r, r0_r):
        xa_r[:, :D_HID] = jnp.dot(x_r[:], wl_r[:],
                                  preferred_element_type=jnp.float32)
        xa_r[:, D_HID:D_HID + 1] = jnp.ones((blk, 1), jnp.float32)
        xa_r[:, D_HID + 1:] = jnp.zeros((blk, D0A - D_HID - 1), jnp.float32)
        r0_r[:] = jnp.dot(x_r[:], wr_r[:], preferred_element_type=jnp.float32)

    return pl.pallas_call(
        body,
        grid=grid,
        in_specs=[
            pl.BlockSpec((blk, D_IN), lambda i: (i, 0)),
            pl.BlockSpec((D_IN, D_HID), lambda i: (0, 0)),
            pl.BlockSpec((D_IN, D_HID), lambda i: (0, 0)),
        ],
        out_specs=[
            pl.BlockSpec((blk, D0A), lambda i: (i, 0)),
            pl.BlockSpec((blk, D_HID), lambda i: (i, 0)),
        ],
        out_shape=[
            jax.ShapeDtypeStruct((N1P, D0A), jnp.float32),
            jax.ShapeDtypeStruct((N1P, D_HID), jnp.float32),
        ],
    )(xpad, wl, wr)


def _mid_tc(p0, p1, r0, bl0, wp, wr1):
    blk = 640
    grid = (N1P // blk,)

    def body(p0_r, p1_r, r0_r, bl_r, wp_r, wr_r, xa_r, r1_r):
        cnt = jnp.maximum(p0_r[:, D_HID:D_HID + 1] + p1_r[:, D_HID:D_HID + 1],
                          1.0)
        mean = (p0_r[:, :D_HID] + p1_r[:, :D_HID]) / cnt
        h = jnp.maximum(mean + bl_r[:] + r0_r[:], 0.0)
        xa_r[:, :D_OUT] = jnp.dot(h, wp_r[:], preferred_element_type=jnp.float32)
        xa_r[:, D_OUT:D_OUT + 1] = jnp.ones((blk, 1), jnp.float32)
        xa_r[:, D_OUT + 1:] = jnp.zeros((blk, D1A - D_OUT - 1), jnp.float32)
        r1_r[:] = jnp.dot(h, wr_r[:], preferred_element_type=jnp.float32)

    return pl.pallas_call(
        body,
        grid=grid,
        in_specs=[
            pl.BlockSpec((blk, D0A), lambda i: (i, 0)),
            pl.BlockSpec((blk, D0A), lambda i: (i, 0)),
            pl.BlockSpec((blk, D_HID), lambda i: (i, 0)),
            pl.BlockSpec((1, D_HID), lambda i: (0, 0)),
            pl.BlockSpec((D_HID, D_OUT), lambda i: (0, 0)),
            pl.BlockSpec((D_HID, D_OUT), lambda i: (0, 0)),
        ],
        out_specs=[
            pl.BlockSpec((blk, D1A), lambda i: (i, 0)),
            pl.BlockSpec((blk, D_OUT), lambda i: (i, 0)),
        ],
        out_shape=[
            jax.ShapeDtypeStruct((N1P, D1A), jnp.float32),
            jax.ShapeDtypeStruct((N1P, D_OUT), jnp.float32),
        ],
    )(p0, p1, r0, bl0, wp, wr1)


def _final_tc(p0, p1, r1, bl1):
    blk = 640
    grid = (N2P // blk,)

    def body(p0_r, p1_r, r1_r, bl_r, o_r):
        cnt = jnp.maximum(p0_r[:, D_OUT:D_OUT + 1] + p1_r[:, D_OUT:D_OUT + 1],
                          1.0)
        mean = (p0_r[:, :D_OUT] + p1_r[:, :D_OUT]) / cnt
        z = mean + bl_r[:] + r1_r[:]
        z = z - jnp.max(z, axis=-1, keepdims=True)
        o_r[:] = z - jnp.log(jnp.sum(jnp.exp(z), axis=-1, keepdims=True))

    return pl.pallas_call(
        body,
        grid=grid,
        in_specs=[
            pl.BlockSpec((blk, D1A), lambda i: (i, 0)),
            pl.BlockSpec((blk, D1A), lambda i: (i, 0)),
            pl.BlockSpec((blk, D_OUT), lambda i: (i, 0)),
            pl.BlockSpec((1, D_OUT), lambda i: (0, 0)),
        ],
        out_specs=pl.BlockSpec((blk, D_OUT), lambda i: (i, 0)),
        out_shape=jax.ShapeDtypeStruct((N2P, D_OUT), jnp.float32),
    )(p0, p1, r1, bl1)


def kernel(x, edge_index0, edge_index1, W_l0, b_l0, W_r0, W_l1, b_l1, W_r1):
    src0, dst0 = _prep_edges(edge_index0, NCH0, CH0, N1, N1, N1P)
    src1, dst1 = _prep_edges(edge_index1, NCH1, CH1, N2, N2, N2P)

    z0 = jnp.zeros((N1P, D0A), jnp.float32)
    z1 = jnp.zeros((N2P, D1A), jnp.float32)

    xpad = jnp.zeros((N1P, D_IN), jnp.float32).at[:N1].set(x[:N1])
    xa0, r0 = _proj0_tc(xpad, W_l0.T, W_r0.T)

    agg0 = _make_agg(N1P, D0A, NCH0, CH0)
    sums0 = agg0(xa0, src0, dst0, z0)

    xa1, r1 = _mid_tc(sums0[0], sums0[1], r0, b_l0.reshape(1, -1),
                      W_l1.T, W_r1.T)

    agg1 = _make_agg(N2P, D1A, NCH1, CH1)
    sums1 = agg1(xa1, src1, dst1, z1)

    out = _final_tc(sums1[0], sums1[1], r1[:N2P], b_l1.reshape(1, -1))
    return out[:N2]


# speedup vs baseline: 7.9079x; 1.1466x over previous
import functools

import jax
import jax.numpy as jnp
from jax import lax
from jax.experimental import pallas as pl
from jax.experimental.pallas import tpu as pltpu
from jax.experimental.pallas import tpu_sc as plsc

N0, N1, N2 = 10000, 5000, 2500
D_IN, D_HID, D_OUT = 256, 256, 128
E0, E1 = 160000, 80000

NC, NS = 2, 16
NW = NC * NS

N1P = 5024
N2P = 2560
D0A = 272
D1A = 144
CH0, NCH0 = 64, 80
CH1, NCH1 = 128, 20
BLK0 = 1256
BLK1 = 640


def _make_agg(n_tgt_pad, d, n_chunks, chunk):
    mesh = plsc.VectorSubcoreMesh(core_axis_name="c", subcore_axis_name="s")
    rz = n_tgt_pad // NS

    @functools.partial(
        pl.kernel,
        mesh=mesh,
        out_type=jax.ShapeDtypeStruct((NC, n_tgt_pad, d), jnp.float32),
        compiler_params=pltpu.CompilerParams(use_tc_tiling_on_sc=False),
        scratch_types=[
            pltpu.VMEM((n_chunks, chunk), jnp.int32),
            pltpu.VMEM((n_chunks, chunk), jnp.int32),
            pltpu.VMEM((chunk, d), jnp.float32),
            pltpu.VMEM((chunk, d), jnp.float32),
            pltpu.VMEM_SHARED((n_tgt_pad, d), jnp.float32),
            pltpu.SemaphoreType.DMA,
        ],
    )
    def agg(table, src, dst, zeros, sums, src_v, dst_v, rows0, rows1, sum_sh,
            sem):
        c = lax.axis_index("c")
        s = lax.axis_index("s")
        wid = c * NS + s
        pltpu.sync_copy(src.at[wid], src_v)
        pltpu.sync_copy(dst.at[wid], dst_v)
        pltpu.sync_copy(zeros.at[pl.ds(s * rz, rz)], sum_sh.at[pl.ds(s * rz, rz)])
        plsc.subcore_barrier()

        pltpu.async_copy(table.at[src_v.at[0]], rows0, sem)

        def pair_body(jj, carry):
            for parity, cur, nxt in ((0, rows0, rows1), (1, rows1, rows0)):
                j = 2 * jj + parity
                jn = (j + 1) % n_chunks
                pltpu.async_copy(table.at[src_v.at[jn]], nxt, sem)
                pltpu.make_async_copy(table.at[src_v.at[j]], cur, sem).wait()
                pltpu.sync_copy(cur, sum_sh.at[dst_v.at[j]], add=True)
            return carry

        lax.fori_loop(0, n_chunks // 2, pair_body, 0)
        pltpu.make_async_copy(table.at[src_v.at[0]], rows0, sem).wait()
        plsc.subcore_barrier()
        pltpu.sync_copy(sum_sh.at[pl.ds(s * rz, rz)], sums.at[c, pl.ds(s * rz, rz)])

    return agg


def _prep_edges(ei, n_chunks, chunk, n_src_mod, n_tgt, n_tgt_pad):
    e = ei.shape[1]
    e_pad = NW * n_chunks * chunk
    ar = jnp.arange(e_pad - e, dtype=jnp.int32)
    src = jnp.concatenate([ei[0].astype(jnp.int32), ar % n_src_mod])
    dst = jnp.concatenate([ei[1].astype(jnp.int32), n_tgt + ar % (n_tgt_pad - n_tgt)])
    return src.reshape(NW, n_chunks, chunk), dst.reshape(NW, n_chunks, chunk)


def _proj0_tc(x, wl, wr):
    blk = BLK0
    grid = (N1P // blk,)

    def body(x_r, wl_r, wr_r, xa_r, r0_r):
        xa_r[:, :D_HID] = jnp.dot(x_r[:], wl_r[:],
                                  preferred_element_type=jnp.float32)
        xa_r[:, D_HID:D_HID + 1] = jnp.ones((blk, 1), jnp.float32)
        xa_r[:, D_HID + 1:] = jnp.zeros((blk, D0A - D_HID - 1), jnp.float32)
        r0_r[:] = jnp.dot(x_r[:], wr_r[:], preferred_element_type=jnp.float32)

    return pl.pallas_call(
        body,
        grid=grid,
        in_specs=[
            pl.BlockSpec((blk, D_IN), lambda i: (i, 0)),
            pl.BlockSpec((D_IN, D_HID), lambda i: (0, 0)),
            pl.BlockSpec((D_IN, D_HID), lambda i: (0, 0)),
        ],
        out_specs=[
            pl.BlockSpec((blk, D0A), lambda i: (i, 0)),
            pl.BlockSpec((blk, D_HID), lambda i: (i, 0)),
        ],
        out_shape=[
            jax.ShapeDtypeStruct((N1P, D0A), jnp.float32),
            jax.ShapeDtypeStruct((N1P, D_HID), jnp.float32),
        ],
    )(x, wl, wr)


def _mid_tc(p0, p1, r0, bl0, wp, wr1):
    blk = BLK0
    grid = (N1P // blk,)

    def body(p0_r, p1_r, r0_r, bl_r, wp_r, wr_r, xa_r, r1_r):
        cnt = jnp.maximum(p0_r[:, D_HID:D_HID + 1] + p1_r[:, D_HID:D_HID + 1],
                          1.0)
        mean = (p0_r[:, :D_HID] + p1_r[:, :D_HID]) / cnt
        h = jnp.maximum(mean + bl_r[:] + r0_r[:], 0.0)
        xa_r[:, :D_OUT] = jnp.dot(h, wp_r[:], preferred_element_type=jnp.float32)
        xa_r[:, D_OUT:D_OUT + 1] = jnp.ones((blk, 1), jnp.float32)
        xa_r[:, D_OUT + 1:] = jnp.zeros((blk, D1A - D_OUT - 1), jnp.float32)
        r1_r[:] = jnp.dot(h, wr_r[:], preferred_element_type=jnp.float32)

    return pl.pallas_call(
        body,
        grid=grid,
        in_specs=[
            pl.BlockSpec((blk, D0A), lambda i: (i, 0)),
            pl.BlockSpec((blk, D0A), lambda i: (i, 0)),
            pl.BlockSpec((blk, D_HID), lambda i: (i, 0)),
            pl.BlockSpec((1, D_HID), lambda i: (0, 0)),
            pl.BlockSpec((D_HID, D_OUT), lambda i: (0, 0)),
            pl.BlockSpec((D_HID, D_OUT), lambda i: (0, 0)),
        ],
        out_specs=[
            pl.BlockSpec((blk, D1A), lambda i: (i, 0)),
            pl.BlockSpec((blk, D_OUT), lambda i: (i, 0)),
        ],
        out_shape=[
            jax.ShapeDtypeStruct((N1P, D1A), jnp.float32),
            jax.ShapeDtypeStruct((N1P, D_OUT), jnp.float32),
        ],
    )(p0, p1, r0, bl0, wp, wr1)


def _final_tc(p0, p1, r1, bl1):
    blk = BLK1
    grid = (N2P // blk,)

    def body(p0_r, p1_r, r1_r, bl_r, o_r):
        cnt = jnp.maximum(p0_r[:, D_OUT:D_OUT + 1] + p1_r[:, D_OUT:D_OUT + 1],
                          1.0)
        mean = (p0_r[:, :D_OUT] + p1_r[:, :D_OUT]) / cnt
        z = mean + bl_r[:] + r1_r[:]
        z = z - jnp.max(z, axis=-1, keepdims=True)
        o_r[:] = z - jnp.log(jnp.sum(jnp.exp(z), axis=-1, keepdims=True))

    return pl.pallas_call(
        body,
        grid=grid,
        in_specs=[
            pl.BlockSpec((blk, D1A), lambda i: (i, 0)),
            pl.BlockSpec((blk, D1A), lambda i: (i, 0)),
            pl.BlockSpec((blk, D_OUT), lambda i: (i, 0)),
            pl.BlockSpec((1, D_OUT), lambda i: (0, 0)),
        ],
        out_specs=pl.BlockSpec((blk, D_OUT), lambda i: (i, 0)),
        out_shape=jax.ShapeDtypeStruct((N2P, D_OUT), jnp.float32),
    )(p0, p1, r1, bl1)


def kernel(x, edge_index0, edge_index1, W_l0, b_l0, W_r0, W_l1, b_l1, W_r1):
    src0, dst0 = _prep_edges(edge_index0, NCH0, CH0, N1, N1, N1P)
    src1, dst1 = _prep_edges(edge_index1, NCH1, CH1, N2, N2, N2P)

    z0 = jnp.zeros((N1P, D0A), jnp.float32)
    z1 = jnp.zeros((N2P, D1A), jnp.float32)

    xa0, r0 = _proj0_tc(x, W_l0.T, W_r0.T)

    agg0 = _make_agg(N1P, D0A, NCH0, CH0)
    sums0 = agg0(xa0, src0, dst0, z0)

    xa1, r1 = _mid_tc(sums0[0], sums0[1], r0, b_l0.reshape(1, -1),
                      W_l1.T, W_r1.T)

    agg1 = _make_agg(N2P, D1A, NCH1, CH1)
    sums1 = agg1(xa1, src1, dst1, z1)

    out = _final_tc(sums1[0], sums1[1], r1, b_l1.reshape(1, -1))
    return out[:N2]


# speedup vs baseline: 7.9924x; 1.0107x over previous
import functools

import jax
import jax.numpy as jnp
from jax import lax
from jax.experimental import pallas as pl
from jax.experimental.pallas import tpu as pltpu
from jax.experimental.pallas import tpu_sc as plsc

N0, N1, N2 = 10000, 5000, 2500
D_IN, D_HID, D_OUT = 256, 256, 128
E0, E1 = 160000, 80000

NC, NS = 2, 16
NW = NC * NS

N1P = 5024
N2P = 2560
D0A = 272
D1A = 144
CH0, NCH0 = 64, 80
CH1, NCH1 = 128, 20
BLK0 = 1256
BLK1 = 640


def _make_agg(n_tgt_pad, d, n_chunks, chunk):
    mesh = plsc.VectorSubcoreMesh(core_axis_name="c", subcore_axis_name="s")
    rz = n_tgt_pad // NS

    @functools.partial(
        pl.kernel,
        mesh=mesh,
        out_type=jax.ShapeDtypeStruct((NC, n_tgt_pad, d), jnp.float32),
        compiler_params=pltpu.CompilerParams(use_tc_tiling_on_sc=False),
        scratch_types=[
            pltpu.VMEM((n_chunks, chunk), jnp.int32),
            pltpu.VMEM((n_chunks, chunk), jnp.int32),
            pltpu.VMEM((chunk, d), jnp.float32),
            pltpu.VMEM((chunk, d), jnp.float32),
            pltpu.VMEM_SHARED((n_tgt_pad, d), jnp.float32),
            pltpu.SemaphoreType.DMA,
        ],
    )
    def agg(table, src, dst, zeros, sums, src_v, dst_v, rows0, rows1, sum_sh,
            sem):
        c = lax.axis_index("c")
        s = lax.axis_index("s")
        wid = c * NS + s
        pltpu.sync_copy(src.at[wid], src_v)
        pltpu.sync_copy(dst.at[wid], dst_v)
        pltpu.sync_copy(zeros.at[pl.ds(s * rz, rz)], sum_sh.at[pl.ds(s * rz, rz)])
        plsc.subcore_barrier()

        pltpu.async_copy(table.at[src_v.at[0]], rows0, sem)

        def pair_body(jj, carry):
            for parity, cur, nxt in ((0, rows0, rows1), (1, rows1, rows0)):
                j = 2 * jj + parity
                jn = (j + 1) % n_chunks
                pltpu.async_copy(table.at[src_v.at[jn]], nxt, sem)
                pltpu.make_async_copy(table.at[src_v.at[j]], cur, sem).wait()
                pltpu.sync_copy(cur, sum_sh.at[dst_v.at[j]], add=True)
            return carry

        lax.fori_loop(0, n_chunks // 2, pair_body, 0)
        pltpu.make_async_copy(table.at[src_v.at[0]], rows0, sem).wait()
        plsc.subcore_barrier()
        pltpu.sync_copy(sum_sh.at[pl.ds(s * rz, rz)], sums.at[c, pl.ds(s * rz, rz)])

    return agg


def _prep_edges(ei, n_chunks, chunk, n_src_mod, n_tgt, n_tgt_pad):
    e = ei.shape[1]
    e_pad = NW * n_chunks * chunk
    ar = jnp.arange(e_pad - e, dtype=jnp.int32)
    src = jnp.concatenate([ei[0].astype(jnp.int32), ar % n_src_mod])
    dst = jnp.concatenate([ei[1].astype(jnp.int32), n_tgt + ar % (n_tgt_pad - n_tgt)])
    return src.reshape(NW, n_chunks, chunk), dst.reshape(NW, n_chunks, chunk)


def _proj0_tc(x, wl):
    blk = BLK0
    grid = (N1P // blk,)

    def body(x_r, wl_r, xa_r):
        xa_r[:, :D_HID] = jnp.dot(x_r[:], wl_r[:],
                                  preferred_element_type=jnp.float32)
        xa_r[:, D_HID:D_HID + 1] = jnp.ones((blk, 1), jnp.float32)
        xa_r[:, D_HID + 1:] = jnp.zeros((blk, D0A - D_HID - 1), jnp.float32)

    return pl.pallas_call(
        body,
        grid=grid,
        in_specs=[
            pl.BlockSpec((blk, D_IN), lambda i: (i, 0)),
            pl.BlockSpec((D_IN, D_HID), lambda i: (0, 0)),
        ],
        out_specs=pl.BlockSpec((blk, D0A), lambda i: (i, 0)),
        out_shape=jax.ShapeDtypeStruct((N1P, D0A), jnp.float32),
    )(x, wl)


def _mid_tc(p0, p1, x, wr0, bl0, wp):
    blk = BLK0
    grid = (N1P // blk,)

    def body(p0_r, p1_r, x_r, wr_r, bl_r, wp_r, xa_r, h_r):
        cnt = jnp.maximum(p0_r[:, D_HID:D_HID + 1] + p1_r[:, D_HID:D_HID + 1],
                          1.0)
        mean = (p0_r[:, :D_HID] + p1_r[:, :D_HID]) / cnt
        r0 = jnp.dot(x_r[:], wr_r[:], preferred_element_type=jnp.float32)
        h = jnp.maximum(mean + bl_r[:] + r0, 0.0)
        xa_r[:, :D_OUT] = jnp.dot(h, wp_r[:], preferred_element_type=jnp.float32)
        xa_r[:, D_OUT:D_OUT + 1] = jnp.ones((blk, 1), jnp.float32)
        xa_r[:, D_OUT + 1:] = jnp.zeros((blk, D1A - D_OUT - 1), jnp.float32)
        h_r[:] = h

    return pl.pallas_call(
        body,
        grid=grid,
        in_specs=[
            pl.BlockSpec((blk, D0A), lambda i: (i, 0)),
            pl.BlockSpec((blk, D0A), lambda i: (i, 0)),
            pl.BlockSpec((blk, D_IN), lambda i: (i, 0)),
            pl.BlockSpec((D_IN, D_HID), lambda i: (0, 0)),
            pl.BlockSpec((1, D_HID), lambda i: (0, 0)),
            pl.BlockSpec((D_HID, D_OUT), lambda i: (0, 0)),
        ],
        out_specs=[
            pl.BlockSpec((blk, D1A), lambda i: (i, 0)),
            pl.BlockSpec((blk, D_HID), lambda i: (i, 0)),
        ],
        out_shape=[
            jax.ShapeDtypeStruct((N1P, D1A), jnp.float32),
            jax.ShapeDtypeStruct((N1P, D_HID), jnp.float32),
        ],
    )(p0, p1, x, wr0, bl0, wp)


def _final_tc(p0, p1, h, wr1, bl1):
    blk = BLK1
    grid = (N2P // blk,)

    def body(p0_r, p1_r, h_r, wr_r, bl_r, o_r):
        cnt = jnp.maximum(p0_r[:, D_OUT:D_OUT + 1] + p1_r[:, D_OUT:D_OUT + 1],
                          1.0)
        mean = (p0_r[:, :D_OUT] + p1_r[:, :D_OUT]) / cnt
        z = mean + bl_r[:] + jnp.dot(h_r[:], wr_r[:],
                                     preferred_element_type=jnp.float32)
        z = z - jnp.max(z, axis=-1, keepdims=True)
        o_r[:] = z - jnp.log(jnp.sum(jnp.exp(z), axis=-1, keepdims=True))

    return pl.pallas_call(
        body,
        grid=grid,
        in_specs=[
            pl.BlockSpec((blk, D1A), lambda i: (i, 0)),
            pl.BlockSpec((blk, D1A), lambda i: (i, 0)),
            pl.BlockSpec((blk, D_HID), lambda i: (i, 0)),
            pl.BlockSpec((D_HID, D_OUT), lambda i: (0, 0)),
            pl.BlockSpec((1, D_OUT), lambda i: (0, 0)),
        ],
        out_specs=pl.BlockSpec((blk, D_OUT), lambda i: (i, 0)),
        out_shape=jax.ShapeDtypeStruct((N2P, D_OUT), jnp.float32),
    )(p0, p1, h, wr1, bl1)


def kernel(x, edge_index0, edge_index1, W_l0, b_l0, W_r0, W_l1, b_l1, W_r1):
    src0, dst0 = _prep_edges(edge_index0, NCH0, CH0, N1, N1, N1P)
    src1, dst1 = _prep_edges(edge_index1, NCH1, CH1, N2, N2, N2P)

    z0 = jnp.zeros((N1P, D0A), jnp.float32)
    z1 = jnp.zeros((N2P, D1A), jnp.float32)

    xa0 = _proj0_tc(x, W_l0.T)

    agg0 = _make_agg(N1P, D0A, NCH0, CH0)
    sums0 = agg0(xa0, src0, dst0, z0)

    xa1, h = _mid_tc(sums0[0], sums0[1], x, W_r0.T, b_l0.reshape(1, -1),
                     W_l1.T)

    agg1 = _make_agg(N2P, D1A, NCH1, CH1)
    sums1 = agg1(xa1, src1, dst1, z1)

    out = _final_tc(sums1[0], sums1[1], h, W_r1.T, b_l1.reshape(1, -1))
    return out[:N2]


# speedup vs baseline: 8.3305x; 1.0423x over previous
import functools

import jax
import jax.numpy as jnp
from jax import lax
from jax.experimental import pallas as pl
from jax.experimental.pallas import tpu as pltpu
from jax.experimental.pallas import tpu_sc as plsc

N0, N1, N2 = 10000, 5000, 2500
D_IN, D_HID, D_OUT = 256, 256, 128
E0, E1 = 160000, 80000

NC, NS = 2, 16
NW = NC * NS

N1P = 5024
N2P = 2560
D0A = 272
D1A = 144
CH0, NCH0 = 64, 80
CH1, NCH1 = 128, 20
BLK0 = 1256
BLK1 = 640


def _make_agg(n_tgt_pad, d, n_chunks, chunk):
    mesh = plsc.VectorSubcoreMesh(core_axis_name="c", subcore_axis_name="s")
    rz = n_tgt_pad // NS

    @functools.partial(
        pl.kernel,
        mesh=mesh,
        out_type=jax.ShapeDtypeStruct((NC, n_tgt_pad, d), jnp.float32),
        compiler_params=pltpu.CompilerParams(use_tc_tiling_on_sc=False),
        scratch_types=[
            pltpu.VMEM((n_chunks, chunk), jnp.int32),
            pltpu.VMEM((n_chunks, chunk), jnp.int32),
            pltpu.VMEM((chunk, d), jnp.float32),
            pltpu.VMEM((chunk, d), jnp.float32),
            pltpu.VMEM_SHARED((n_tgt_pad, d), jnp.float32),
            pltpu.SemaphoreType.DMA,
        ],
    )
    def agg(table, src, dst, zeros, sums, src_v, dst_v, rows0, rows1, sum_sh,
            sem):
        c = lax.axis_index("c")
        s = lax.axis_index("s")
        wid = c * NS + s
        pltpu.sync_copy(src.at[wid], src_v)
        pltpu.sync_copy(dst.at[wid], dst_v)
        pltpu.sync_copy(zeros.at[pl.ds(s * rz, rz)], sum_sh.at[pl.ds(s * rz, rz)])
        plsc.subcore_barrier()

        pltpu.async_copy(table.at[src_v.at[0]], rows0, sem)

        def pair_body(jj, carry):
            for parity, cur, nxt in ((0, rows0, rows1), (1, rows1, rows0)):
                j = 2 * jj + parity
                jn = (j + 1) % n_chunks
                pltpu.async_copy(table.at[src_v.at[jn]], nxt, sem)
                pltpu.make_async_copy(table.at[src_v.at[j]], cur, sem).wait()
                pltpu.sync_copy(cur, sum_sh.at[dst_v.at[j]], add=True)
            return carry

        lax.fori_loop(0, n_chunks // 2, pair_body, 0)
        pltpu.make_async_copy(table.at[src_v.at[0]], rows0, sem).wait()
        plsc.subcore_barrier()
        pltpu.sync_copy(sum_sh.at[pl.ds(s * rz, rz)], sums.at[c, pl.ds(s * rz, rz)])

    return agg


def _prep_edges(ei, n_chunks, chunk, n_src_mod, n_tgt, n_tgt_pad):
    e = ei.shape[1]
    e_pad = NW * n_chunks * chunk
    ar = jnp.arange(e_pad - e, dtype=jnp.int32)
    src = jnp.concatenate([ei[0].astype(jnp.int32), ar % n_src_mod])
    dst = jnp.concatenate([ei[1].astype(jnp.int32), n_tgt + ar % (n_tgt_pad - n_tgt)])
    return src.reshape(NW, n_chunks, chunk), dst.reshape(NW, n_chunks, chunk)


def _proj0_tc(x, wl):
    blk = BLK0
    grid = (N1P // blk,)

    def body(x_r, wl_r, xa_r):
        xa_r[:, :D_HID] = jnp.dot(x_r[:], wl_r[:],
                                  preferred_element_type=jnp.float32)
        xa_r[:, D_HID:D_HID + 1] = jnp.ones((blk, 1), jnp.float32)
        xa_r[:, D_HID + 1:] = jnp.zeros((blk, D0A - D_HID - 1), jnp.float32)

    return pl.pallas_call(
        body,
        grid=grid,
        in_specs=[
            pl.BlockSpec((blk, D_IN), lambda i: (i, 0)),
            pl.BlockSpec((D_IN, D_HID), lambda i: (0, 0)),
        ],
        out_specs=pl.BlockSpec((blk, D0A), lambda i: (i, 0)),
        out_shape=jax.ShapeDtypeStruct((N1P, D0A), jnp.float32),
    )(x, wl)


def _mid_tc(sums, x, wr0, bl0, wp):
    blk = BLK0
    grid = (N1P // blk,)

    def body(p_r, x_r, wr_r, bl_r, wp_r, xa_r, h_r):
        p = p_r[0] + p_r[1]
        cnt = jnp.maximum(p[:, D_HID:D_HID + 1], 1.0)
        mean = p[:, :D_HID] / cnt
        r0 = jnp.dot(x_r[:], wr_r[:], preferred_element_type=jnp.float32)
        h = jnp.maximum(mean + bl_r[:] + r0, 0.0)
        xa_r[:, :D_OUT] = jnp.dot(h, wp_r[:], preferred_element_type=jnp.float32)
        xa_r[:, D_OUT:D_OUT + 1] = jnp.ones((blk, 1), jnp.float32)
        xa_r[:, D_OUT + 1:] = jnp.zeros((blk, D1A - D_OUT - 1), jnp.float32)
        h_r[:] = h

    return pl.pallas_call(
        body,
        grid=grid,
        in_specs=[
            pl.BlockSpec((2, blk, D0A), lambda i: (0, i, 0)),
            pl.BlockSpec((blk, D_IN), lambda i: (i, 0)),
            pl.BlockSpec((D_IN, D_HID), lambda i: (0, 0)),
            pl.BlockSpec((1, D_HID), lambda i: (0, 0)),
            pl.BlockSpec((D_HID, D_OUT), lambda i: (0, 0)),
        ],
        out_specs=[
            pl.BlockSpec((blk, D1A), lambda i: (i, 0)),
            pl.BlockSpec((blk, D_HID), lambda i: (i, 0)),
        ],
        out_shape=[
            jax.ShapeDtypeStruct((N1P, D1A), jnp.float32),
            jax.ShapeDtypeStruct((N1P, D_HID), jnp.float32),
        ],
    )(sums, x, wr0, bl0, wp)


def _final_tc(sums, h, wr1, bl1):
    blk = BLK1
    grid = (N2P // blk,)

    def body(p_r, h_r, wr_r, bl_r, o_r):
        p = p_r[0] + p_r[1]
        cnt = jnp.maximum(p[:, D_OUT:D_OUT + 1], 1.0)
        mean = p[:, :D_OUT] / cnt
        z = mean + bl_r[:] + jnp.dot(h_r[:], wr_r[:],
                                     preferred_element_type=jnp.float32)
        z = z - jnp.max(z, axis=-1, keepdims=True)
        o_r[:] = z - jnp.log(jnp.sum(jnp.exp(z), axis=-1, keepdims=True))

    return pl.pallas_call(
        body,
        grid=grid,
        in_specs=[
            pl.BlockSpec((2, blk, D1A), lambda i: (0, i, 0)),
            pl.BlockSpec((blk, D_HID), lambda i: (i, 0)),
            pl.BlockSpec((D_HID, D_OUT), lambda i: (0, 0)),
            pl.BlockSpec((1, D_OUT), lambda i: (0, 0)),
        ],
        out_specs=pl.BlockSpec((blk, D_OUT), lambda i: (i, 0)),
        out_shape=jax.ShapeDtypeStruct((N2P, D_OUT), jnp.float32),
    )(sums, h, wr1, bl1)


def kernel(x, edge_index0, edge_index1, W_l0, b_l0, W_r0, W_l1, b_l1, W_r1):
    src0, dst0 = _prep_edges(edge_index0, NCH0, CH0, N1, N1, N1P)
    src1, dst1 = _prep_edges(edge_index1, NCH1, CH1, N2, N2, N2P)

    z0 = jnp.zeros((N1P, D0A), jnp.float32)
    z1 = jnp.zeros((N2P, D1A), jnp.float32)

    xa0 = _proj0_tc(x, W_l0.T)

    agg0 = _make_agg(N1P, D0A, NCH0, CH0)
    sums0 = agg0(xa0, src0, dst0, z0)

    xa1, h = _mid_tc(sums0, x, W_r0.T, b_l0.reshape(1, -1), W_l1.T)

    agg1 = _make_agg(N2P, D1A, NCH1, CH1)
    sums1 = agg1(xa1, src1, dst1, z1)

    out = _final_tc(sums1, h, W_r1.T, b_l1.reshape(1, -1))
    return out[:N2]
